# baseline (speedup 1.0000x reference)
import functools

import jax
import jax.numpy as jnp
from jax import lax
from jax.experimental import pallas as pl
from jax.experimental.pallas import tpu as pltpu
from jax.experimental.pallas import tpu_sc as plsc

E = 8
TOPK = 2
CAP = 1024
LANES = 128
BLK = 256
M = 256
NEG = -1e30
NTILES = 32
CHD = 32


def _router_body(x_ref, wr_ref, dst_ref, gsrc_ref, w_ref, counts_ref):
    T = x_ref.shape[0]
    A = TOPK * T
    logits = jnp.dot(x_ref[...], wr_ref[...], preferred_element_type=jnp.float32)
    li = lax.broadcasted_iota(jnp.int32, (T, E), 1)
    m = jnp.max(logits, axis=1, keepdims=True)
    ex = jnp.exp(logits - m)
    probs = ex / jnp.sum(ex, axis=1, keepdims=True)
    v0 = jnp.max(probs, axis=1, keepdims=True)
    i0 = jnp.min(jnp.where(probs == v0, li, E), axis=1, keepdims=True)
    pm1 = jnp.where(li == i0, -1.0, probs)
    v1 = jnp.max(pm1, axis=1, keepdims=True)
    i1 = jnp.min(jnp.where(pm1 == v1, li, E), axis=1, keepdims=True)
    mask0 = (li == i0).astype(jnp.float32)
    mask1 = (li == i1).astype(jnp.float32)

    ri = lax.broadcasted_iota(jnp.int32, (BLK, BLK), 0)
    ci = lax.broadcasted_iota(jnp.int32, (BLK, BLK), 1)
    tri = (ci < ri).astype(jnp.float32)
    mask_all = jnp.concatenate([mask0, mask1], axis=0)
    base = jnp.zeros((1, E), jnp.float32)
    pies = []
    for i in range(A // BLK):
        mb = mask_all[i * BLK:(i + 1) * BLK, :]
        posb = base + jnp.dot(tri, mb, preferred_element_type=jnp.float32)
        pies.append(jnp.sum(posb * mb, axis=1, keepdims=True))
        base = base + jnp.sum(mb, axis=0, keepdims=True)
    pie = jnp.concatenate(pies, axis=0)

    keep = pie < float(CAP)
    p = jnp.minimum(pie, float(CAP - 1)).astype(jnp.int32)
    eidx = jnp.concatenate([i0, i1], axis=0)
    a_iota = lax.broadcasted_iota(jnp.int32, (A, 1), 0)
    dump = E * CAP + a_iota // (A // NTILES)
    flat = jnp.where(keep, eidx * CAP + p, dump)
    val = jnp.concatenate([v0, v1], axis=0)
    dst_ref[...] = flat
    gsrc_ref[...] = jnp.where(keep, flat, E * CAP)
    wv = jnp.where(keep, val, 0.0)
    w_ref[...] = jnp.broadcast_to(wv, (A, LANES))
    counts_ref[...] = jnp.minimum(base, float(CAP)).astype(jnp.int32)


def _router(x2, wr):
    T = x2.shape[0]
    A = TOPK * T
    return pl.pallas_call(
        _router_body,
        out_shape=(
            jax.ShapeDtypeStruct((A, 1), jnp.int32),
            jax.ShapeDtypeStruct((A, 1), jnp.int32),
            jax.ShapeDtypeStruct((A, LANES), jnp.float32),
            jax.ShapeDtypeStruct((1, E), jnp.int32),
        ),
    )(x2, wr)


def _dispatch(x2, dst, wb):
    T, D = x2.shape
    A = dst.shape[0]
    per_tile = A // NTILES
    nrows = E * CAP + NTILES
    wrows = E * CAP + M
    mesh = plsc.VectorSubcoreMesh(core_axis_name="c", subcore_axis_name="s")

    @functools.partial(
        pl.kernel,
        out_type=(
            jax.ShapeDtypeStruct((nrows, D), jnp.float32),
            jax.ShapeDtypeStruct((wrows, LANES), jnp.float32),
        ),
        mesh=mesh,
        scratch_types=[
            pltpu.VMEM((2, CHD), jnp.int32),
            pltpu.VMEM((2, CHD, D), jnp.float32),
            pltpu.VMEM((2, CHD, LANES), jnp.float32),
            pltpu.SemaphoreType.DMA,
            pltpu.SemaphoreType.DMA,
            pltpu.SemaphoreType.DMA,
            pltpu.SemaphoreType.DMA,
        ],
    )
    def k(x_hbm, dst_hbm, wb_hbm, buf_hbm, wslot_hbm, idx_v, rows_v, wv_v,
          s0, s1, s2, s3):
        wid = lax.axis_index("s") * 2 + lax.axis_index("c")
        sems = ((s0, s1), (s2, s3))
        nch = per_tile // CHD
        scats = [None, None]
        for i in range(nch):
            b = i & 1
            if scats[b] is not None:
                for cp in scats[b]:
                    cp.wait()
            base = wid * per_tile + i * CHD
            src = lax.rem(base, T)
            pltpu.sync_copy(dst_hbm.at[pl.ds(base, CHD)], idx_v.at[b])
            pltpu.sync_copy(x_hbm.at[pl.ds(src, CHD)], rows_v.at[b])
            pltpu.sync_copy(wb_hbm.at[pl.ds(base, CHD)], wv_v.at[b])
            scats[b] = (
                pltpu.async_copy(rows_v.at[b], buf_hbm.at[idx_v.at[b]],
                                 sems[b][0]),
                pltpu.async_copy(wv_v.at[b], wslot_hbm.at[idx_v.at[b]],
                                 sems[b][1]),
            )
        for pair in scats:
            for cp in pair:
                cp.wait()

    return k(x2, dst, wb)


def _ffn_body(cnts, xb, w1, w2, ws, ob):
    e = pl.program_id(0)
    cb = pl.program_id(1)
    cnt = cnts[jnp.minimum(e, E - 1)]

    @pl.when(jnp.logical_and(e < E, cb * M < cnt))
    def _():
        h = jnp.dot(xb[...], w1[0], preferred_element_type=jnp.float32)
        h = jax.nn.gelu(h)
        out = jnp.dot(h, w2[0], preferred_element_type=jnp.float32)
        ob[...] = out * ws[...][:, :1]

    @pl.when(e == E)
    def _():
        ob[...] = jnp.zeros(ob.shape, ob.dtype)


def _ffn(buf, W1, W2, counts, wslot):
    _, D = buf.shape
    F = W1.shape[2]
    nb_cap = CAP // M

    def im_x(e, cb, cnts):
        ee = jnp.minimum(e, E - 1)
        nb = (cnts[ee] + (M - 1)) // M
        last = jnp.maximum(nb - 1, 0)
        cbe = jnp.where(e < E, jnp.minimum(cb, last), last)
        return (ee * nb_cap + cbe, 0)

    def im_out(e, cb, cnts):
        nb = (cnts[jnp.minimum(e, E - 1)] + (M - 1)) // M
        cbe = jnp.minimum(cb, jnp.maximum(nb - 1, 0))
        return (jnp.where(e < E, e * nb_cap + cbe, E * nb_cap), 0)

    def im_w(e, cb, cnts):
        return (jnp.minimum(e, E - 1), 0, 0)

    grid_spec = pltpu.PrefetchScalarGridSpec(
        num_scalar_prefetch=1,
        grid=(E + 1, nb_cap),
        in_specs=[
            pl.BlockSpec((M, D), im_x),
            pl.BlockSpec((1, D, F), im_w),
            pl.BlockSpec((1, F, D), im_w),
            pl.BlockSpec((M, LANES), im_out),
        ],
        out_specs=pl.BlockSpec((M, D), im_out),
    )
    return pl.pallas_call(
        _ffn_body,
        grid_spec=grid_spec,
        out_shape=jax.ShapeDtypeStruct((E * CAP + M, D), jnp.float32),
        compiler_params=pltpu.CompilerParams(
            dimension_semantics=("arbitrary", "arbitrary")),
    )(counts, buf, W1, W2, wslot)


def _gather_combine(ob, gsrc):
    _, D = ob.shape
    A = gsrc.shape[0]
    T = A // TOPK
    ntok = T // NTILES
    TOK = 16
    mesh = plsc.VectorSubcoreMesh(core_axis_name="c", subcore_axis_name="s")

    @functools.partial(
        pl.kernel,
        out_type=jax.ShapeDtypeStruct((T, D), jnp.float32),
        mesh=mesh,
        scratch_types=[
            pltpu.VMEM((2, TOK), jnp.int32),
            pltpu.VMEM((2, TOK), jnp.int32),
            pltpu.VMEM((2, TOK, D), jnp.float32),
            pltpu.VMEM((2, TOK, D), jnp.float32),
            pltpu.VMEM((TOK, D), jnp.float32),
            pltpu.SemaphoreType.DMA,
            pltpu.SemaphoreType.DMA,
            pltpu.SemaphoreType.DMA,
            pltpu.SemaphoreType.DMA,
        ],
    )
    def k(ob_hbm, gsrc_hbm, y_hbm, idx0_v, idx1_v, r0_v, r1_v, y_v,
          s0, s1, s2, s3):
        wid = lax.axis_index("s") * 2 + lax.axis_index("c")
        sems = ((s0, s1), (s2, s3))
        nch = ntok // TOK

        def start_gather(i, b):
            t0 = wid * ntok + i * TOK
            pltpu.sync_copy(gsrc_hbm.at[pl.ds(t0, TOK)], idx0_v.at[b])
            pltpu.sync_copy(gsrc_hbm.at[pl.ds(T + t0, TOK)], idx1_v.at[b])
            return (
                pltpu.async_copy(ob_hbm.at[idx0_v.at[b]], r0_v.at[b],
                                 sems[b][0]),
                pltpu.async_copy(ob_hbm.at[idx1_v.at[b]], r1_v.at[b],
                                 sems[b][1]),
            )

        pend = [None, None]
        pend[0] = start_gather(0, 0)
        for i in range(nch):
            b = i & 1
            for cp in pend[b]:
                cp.wait()
            if i + 1 < nch:
                pend[(i + 1) & 1] = start_gather(i + 1, (i + 1) & 1)

            def tok_body(i_, carry):
                for j in range(D // 16):
                    sl = pl.ds(j * 16, 16)
                    y_v[i_, sl] = r0_v[b, i_, sl] + r1_v[b, i_, sl]
                return carry

            lax.fori_loop(0, TOK, tok_body, 0)
            t0 = wid * ntok + i * TOK
            pltpu.sync_copy(y_v, y_hbm.at[pl.ds(t0, TOK)])

    return k(ob, gsrc)


def kernel(x, W_r, W1, W2):
    B, S, D = x.shape
    T = B * S
    x2 = x.reshape(T, D)
    dst, gsrc, w, counts2 = _router(x2, W_r)
    counts = counts2.reshape(E)
    buf, wslot = _dispatch(x2, dst.reshape(-1), w)
    ob = _ffn(buf, W1, W2, counts, wslot)
    y = _gather_combine(ob, gsrc.reshape(-1))
    return y.reshape(B, S, D)


# speedup vs baseline: 1.0317x; 1.0317x over previous
import functools

import jax
import jax.numpy as jnp
from jax import lax
from jax.experimental import pallas as pl
from jax.experimental.pallas import tpu as pltpu
from jax.experimental.pallas import tpu_sc as plsc

E = 8
TOPK = 2
CAP = 1024
LANES = 128
BLK = 256
M = 256
NEG = -1e30
NTILES = 32
CH = 64


def _router_body(x_ref, wr_ref, dst_ref, gsrc_ref, w_ref, counts_ref):
    T = x_ref.shape[0]
    A = TOPK * T
    logits = jnp.dot(x_ref[...], wr_ref[...], preferred_element_type=jnp.float32)
    li = lax.broadcasted_iota(jnp.int32, (T, E), 1)
    m = jnp.max(logits, axis=1, keepdims=True)
    ex = jnp.exp(logits - m)
    probs = ex / jnp.sum(ex, axis=1, keepdims=True)
    v0 = jnp.max(probs, axis=1, keepdims=True)
    i0 = jnp.min(jnp.where(probs == v0, li, E), axis=1, keepdims=True)
    pm1 = jnp.where(li == i0, -1.0, probs)
    v1 = jnp.max(pm1, axis=1, keepdims=True)
    i1 = jnp.min(jnp.where(pm1 == v1, li, E), axis=1, keepdims=True)
    mask0 = (li == i0).astype(jnp.float32)
    mask1 = (li == i1).astype(jnp.float32)

    ri = lax.broadcasted_iota(jnp.int32, (BLK, BLK), 0)
    ci = lax.broadcasted_iota(jnp.int32, (BLK, BLK), 1)
    tri = (ci < ri).astype(jnp.float32)
    mask_all = jnp.concatenate([mask0, mask1], axis=0)
    base = jnp.zeros((1, E), jnp.float32)
    pies = []
    for i in range(A // BLK):
        mb = mask_all[i * BLK:(i + 1) * BLK, :]
        posb = base + jnp.dot(tri, mb, preferred_element_type=jnp.float32)
        pies.append(jnp.sum(posb * mb, axis=1, keepdims=True))
        base = base + jnp.sum(mb, axis=0, keepdims=True)
    pie = jnp.concatenate(pies, axis=0)

    keep = pie < float(CAP)
    p = jnp.minimum(pie, float(CAP - 1)).astype(jnp.int32)
    eidx = jnp.concatenate([i0, i1], axis=0)
    a_iota = lax.broadcasted_iota(jnp.int32, (A, 1), 0)
    dump = E * CAP + a_iota // (A // NTILES)
    flat = jnp.where(keep, eidx * CAP + p, dump)
    val = jnp.concatenate([v0, v1], axis=0)
    dst_ref[...] = flat
    gsrc_ref[...] = jnp.minimum(flat, E * CAP - 1)
    w_ref[...] = jnp.where(keep, val, 0.0)
    counts_ref[...] = jnp.minimum(base, float(CAP)).astype(jnp.int32)


def _router(x2, wr):
    T = x2.shape[0]
    A = TOPK * T
    return pl.pallas_call(
        _router_body,
        out_shape=(
            jax.ShapeDtypeStruct((A, 1), jnp.int32),
            jax.ShapeDtypeStruct((A, 1), jnp.int32),
            jax.ShapeDtypeStruct((A, 1), jnp.float32),
            jax.ShapeDtypeStruct((1, E), jnp.int32),
        ),
    )(x2, wr)


def _dispatch(x2, dst):
    T, D = x2.shape
    A = dst.shape[0]
    per_tile = A // NTILES
    nrows = E * CAP + NTILES
    mesh = plsc.VectorSubcoreMesh(core_axis_name="c", subcore_axis_name="s")

    @functools.partial(
        pl.kernel,
        out_type=jax.ShapeDtypeStruct((nrows, D), jnp.float32),
        mesh=mesh,
        scratch_types=[
            pltpu.VMEM((CH,), jnp.int32),
            pltpu.VMEM((CH, D), jnp.float32),
            pltpu.SemaphoreType.DMA,
        ],
    )
    def k(x_hbm, dst_hbm, buf_hbm, idx_v, rows_v, sem):
        wid = lax.axis_index("s") * 2 + lax.axis_index("c")
        for i in range(per_tile // CH):
            base = wid * per_tile + i * CH
            src = lax.rem(base, T)
            pltpu.sync_copy(dst_hbm.at[pl.ds(base, CH)], idx_v)
            pltpu.sync_copy(x_hbm.at[pl.ds(src, CH)], rows_v)
            pltpu.async_copy(rows_v, buf_hbm.at[idx_v], sem).wait()

    return k(x2, dst)


def _ffn_body(cnts, xb, w1, w2, ob):
    cb = pl.program_id(1)
    cnt = cnts[pl.program_id(0)]

    @pl.when(cb * M < cnt)
    def _():
        h = jnp.dot(xb[...], w1[0], preferred_element_type=jnp.float32)
        h = jax.nn.gelu(h)
        ob[...] = jnp.dot(h, w2[0], preferred_element_type=jnp.float32)


def _ffn(buf, W1, W2, counts):
    _, D = buf.shape
    F = W1.shape[2]
    nb_cap = CAP // M

    def im_io(e, cb, cnts):
        nb = (cnts[e] + (M - 1)) // M
        cbe = jnp.minimum(cb, jnp.maximum(nb - 1, 0))
        return (e * nb_cap + cbe, 0)

    grid_spec = pltpu.PrefetchScalarGridSpec(
        num_scalar_prefetch=1,
        grid=(E, nb_cap),
        in_specs=[
            pl.BlockSpec((M, D), im_io),
            pl.BlockSpec((1, D, F), lambda e, cb, c: (e, 0, 0)),
            pl.BlockSpec((1, F, D), lambda e, cb, c: (e, 0, 0)),
        ],
        out_specs=pl.BlockSpec((M, D), im_io),
    )
    return pl.pallas_call(
        _ffn_body,
        grid_spec=grid_spec,
        out_shape=jax.ShapeDtypeStruct((E * CAP, D), jnp.float32),
        compiler_params=pltpu.CompilerParams(
            dimension_semantics=("arbitrary", "arbitrary")),
    )(counts, buf, W1, W2)


def _gather(ob, gsrc):
    _, D = ob.shape
    A = gsrc.shape[0]
    per_tile = A // NTILES
    mesh = plsc.VectorSubcoreMesh(core_axis_name="c", subcore_axis_name="s")

    @functools.partial(
        pl.kernel,
        out_type=jax.ShapeDtypeStruct((A, D), jnp.float32),
        mesh=mesh,
        scratch_types=[
            pltpu.VMEM((CH,), jnp.int32),
            pltpu.VMEM((CH, D), jnp.float32),
            pltpu.SemaphoreType.DMA,
        ],
    )
    def k(src_hbm, gsrc_hbm, rg_hbm, idx_v, rows_v, sem):
        wid = lax.axis_index("s") * 2 + lax.axis_index("c")
        for i in range(per_tile // CH):
            base = wid * per_tile + i * CH
            pltpu.sync_copy(gsrc_hbm.at[pl.ds(base, CH)], idx_v)
            pltpu.async_copy(src_hbm.at[idx_v], rows_v, sem).wait()
            pltpu.sync_copy(rows_v, rg_hbm.at[pl.ds(base, CH)])

    return k(ob, gsrc)


def _combine_body(r0, r1, w0r, w1r, y):
    a = jnp.where(w0r[...] != 0.0, w0r[...] * r0[...], 0.0)
    b = jnp.where(w1r[...] != 0.0, w1r[...] * r1[...], 0.0)
    y[...] = a + b


def _combine(rg, w):
    A, D = rg.shape
    T = A // 2
    nblk = 4
    rb = T // nblk
    return pl.pallas_call(
        _combine_body,
        grid=(nblk,),
        in_specs=[
            pl.BlockSpec((rb, D), lambda i: (i, 0)),
            pl.BlockSpec((rb, D), lambda i: (i + nblk, 0)),
            pl.BlockSpec((rb, 1), lambda i: (i, 0)),
            pl.BlockSpec((rb, 1), lambda i: (i + nblk, 0)),
        ],
        out_specs=pl.BlockSpec((rb, D), lambda i: (i, 0)),
        out_shape=jax.ShapeDtypeStruct((T, D), jnp.float32),
    )(rg, rg, w, w)


def kernel(x, W_r, W1, W2):
    B, S, D = x.shape
    T = B * S
    x2 = x.reshape(T, D)
    dst, gsrc, w, counts2 = _router(x2, W_r)
    counts = counts2.reshape(E)
    buf = _dispatch(x2, dst.reshape(-1))
    ob = _ffn(buf, W1, W2, counts)
    rg = _gather(ob, gsrc.reshape(-1))
    y = _combine(rg, w)
    return y.reshape(B, S, D)
